# baseline (speedup 1.0000x reference)
import functools

import jax
import jax.numpy as jnp
from jax import lax
from jax.experimental import pallas as pl
from jax.experimental.pallas import tpu as pltpu
from jax.experimental.pallas import tpu_sc as plsc


def _sc_gather_sum(table, flat_idx, n_groups, rows_per_group, d):
    info = plsc.get_sparse_core_info()
    nc = info.num_cores
    mesh = plsc.VectorSubcoreMesh(core_axis_name="c", subcore_axis_name="s")

    @functools.partial(
        pl.kernel,
        mesh=mesh,
        out_type=jax.ShapeDtypeStruct((n_groups, d), jnp.float32),
        scratch_types=[
            pltpu.VMEM((rows_per_group,), jnp.int32),
            pltpu.VMEM((rows_per_group, d), jnp.float32),
            pltpu.VMEM((d,), jnp.float32),
            pltpu.SemaphoreType.DMA,
        ],
    )
    def gather_kernel(table_hbm, idx_hbm, out_hbm, idx_v, rows_v, acc_v, sem):
        wid = lax.axis_index("s") * nc + lax.axis_index("c")

        @pl.when(wid < n_groups)
        def _():
            base = wid * rows_per_group
            pltpu.sync_copy(idx_hbm.at[pl.ds(base, rows_per_group)], idx_v)
            pltpu.async_copy(table_hbm.at[idx_v], rows_v, sem).wait()

            def body(i, carry):
                sl = pl.ds(i * 16, 16)
                acc = rows_v[0, sl]
                for r in range(1, rows_per_group):
                    acc = acc + rows_v[r, sl]
                acc_v[sl] = acc
                return carry

            lax.fori_loop(0, d // 16, body, 0)
            pltpu.sync_copy(acc_v, out_hbm.at[wid])

    return gather_kernel(table, flat_idx)


def _tc_attn(h, e):
    B, S, D = h.shape

    def body(h_ref, e_ref, o_ref):
        hb = h_ref[0]
        eb = e_ref[0]
        s = lax.dot_general(
            hb, eb, (((1,), (1,)), ((), ())),
            preferred_element_type=jnp.float32,
            precision=lax.Precision.HIGHEST,
        )
        m = jnp.max(s, axis=0, keepdims=True)
        p = jnp.exp(s - m)
        z = jnp.sum(p, axis=0, keepdims=True)
        w = p / z
        o_ref[0, 0] = 0.5 * jnp.sum(w, axis=1)

    out = pl.pallas_call(
        body,
        grid=(B,),
        in_specs=[
            pl.BlockSpec((1, S, D), lambda b: (b, 0, 0)),
            pl.BlockSpec((1, 2, D), lambda b: (b, 0, 0)),
        ],
        out_specs=pl.BlockSpec((1, 1, S), lambda b: (b, 0, 0)),
        out_shape=jax.ShapeDtypeStruct((B, 1, S), jnp.float32),
    )(h, e)
    return out[:, 0, :]


def kernel(input_embed_M, e1_index, e2_index):
    B, S, D = input_embed_M.shape
    K = e1_index.shape[-1]
    eidx = jnp.concatenate(
        [e1_index.astype(jnp.int32), e2_index.astype(jnp.int32)], axis=1
    )
    flat_idx = (
        eidx + (jnp.arange(B, dtype=jnp.int32) * S)[:, None]
    ).reshape(B * 2 * K)
    table = input_embed_M.reshape(B * S, D)
    embeds = _sc_gather_sum(table, flat_idx, B * 2, K, D)
    e = embeds.reshape(B, 2, D)
    return _tc_attn(input_embed_M, e)


# speedup vs baseline: 3.3697x; 3.3697x over previous
import functools

import jax
import jax.numpy as jnp
from jax import lax
from jax.experimental import pallas as pl
from jax.experimental.pallas import tpu as pltpu


def _attn_body(idx_ref, h_ref, o_ref):
    K = idx_ref.shape[-1] // 2
    hb = h_ref[0]
    e1 = h_ref[0, idx_ref[0, 0, 0], :]
    e2 = h_ref[0, idx_ref[0, 0, K], :]
    for k in range(1, K):
        e1 = e1 + h_ref[0, idx_ref[0, 0, k], :]
        e2 = e2 + h_ref[0, idx_ref[0, 0, K + k], :]
    e12 = jnp.stack([e1, e2], axis=0)
    s = lax.dot_general(
        hb, e12, (((1,), (1,)), ((), ())),
        preferred_element_type=jnp.float32,
    )
    p = jnp.exp(s - jnp.max(s, axis=0, keepdims=True))
    w = p / jnp.sum(p, axis=0, keepdims=True)
    o_ref[0, 0] = 0.5 * jnp.sum(w, axis=1)


def kernel(input_embed_M, e1_index, e2_index):
    B, S, D = input_embed_M.shape
    K = e1_index.shape[-1]
    eidx = jnp.concatenate(
        [e1_index.astype(jnp.int32), e2_index.astype(jnp.int32)], axis=1
    ).reshape(B, 1, 2 * K)
    out = pl.pallas_call(
        _attn_body,
        grid=(B,),
        in_specs=[
            pl.BlockSpec((1, 1, 2 * K), lambda b: (b, 0, 0), memory_space=pltpu.SMEM),
            pl.BlockSpec((1, S, D), lambda b: (b, 0, 0)),
        ],
        out_specs=pl.BlockSpec((1, 1, S), lambda b: (b, 0, 0)),
        out_shape=jax.ShapeDtypeStruct((B, 1, S), jnp.float32),
    )(eidx, input_embed_M)
    return out[:, 0, :]
